# baseline (speedup 1.0000x reference)
import functools

import jax
import jax.numpy as jnp
from jax import lax
from jax.experimental import pallas as pl
from jax.experimental.pallas import tpu as pltpu
from jax.experimental.pallas import tpu_sc as plsc

NC = 2
NS = 16
L = 16
NW = NC * NS


@functools.lru_cache(maxsize=None)
def _build(B, ITEM_N, USER_N, M, D, C):
    per_w = B // NW
    n_chunks = per_w // C
    n_groups = C // L
    JMAX = M - 2

    mesh = plsc.VectorSubcoreMesh(core_axis_name="c", subcore_axis_name="s")

    @functools.partial(
        pl.kernel,
        out_type=jax.ShapeDtypeStruct((B,), jnp.float32),
        mesh=mesh,
        compiler_params=pltpu.CompilerParams(
            needs_layout_passes=False, use_tc_tiling_on_sc=True,
            skip_device_barrier=True),
        scratch_types=[
            [pltpu.VMEM((C + L,), jnp.int32) for _ in range(2)],
            [pltpu.VMEM((C + L,), jnp.int32) for _ in range(2)],
            [pltpu.VMEM((C, M * D), jnp.float32) for _ in range(2)],
            [pltpu.VMEM((C,), jnp.int32) for _ in range(2)],
            [pltpu.VMEM((C, 2 * D), jnp.float32) for _ in range(2)],
            [pltpu.VMEM((C,), jnp.int32) for _ in range(2)],
            [pltpu.VMEM((C,), jnp.float32) for _ in range(2)],
            [pltpu.SemaphoreType.DMA for _ in range(2)],
            pltpu.SemaphoreType.DMA,
        ],
    )
    def kern(uid_hbm, iid_hbm, users_hbm, items_hbm, nb_hbm, out_hbm,
             iid_v, uid_v, e_v, upix_v, u_v, nb_v, resp_v, sems, osem):
        wid = lax.axis_index("s") * NC + lax.axis_index("c")
        base = wid * per_w

        def fire(c):
            b = c % 2
            off = base + c * C
            pltpu.sync_copy(iid_hbm.at[pl.ds(off, C)],
                            iid_v[b].at[pl.ds(0, C)])
            pltpu.sync_copy(uid_hbm.at[pl.ds(off, C)],
                            uid_v[b].at[pl.ds(0, C)])

            pltpu.async_copy(
                items_hbm.at[iid_v[b].at[pl.ds(0, C)]], e_v[b], sems[b])
            for g in range(n_groups):
                sl = pl.ds(g * L, L)
                upix_v[b][sl] = lax.shift_right_logical(uid_v[b][sl], 1)
            pltpu.async_copy(
                users_hbm.at[upix_v[b]], u_v[b], sems[b])
            return pltpu.async_copy(
                nb_hbm.at[iid_v[b].at[pl.ds(0, C)]], nb_v[b], sems[b])

        def drain(c, nb_h):
            b = c % 2
            pltpu.make_async_copy(
                items_hbm.at[pl.ds(0, C), :], e_v[b], sems[b]).wait()
            pltpu.make_async_copy(
                users_hbm.at[pl.ds(0, C), :], u_v[b], sems[b]).wait()
            nb_h.wait()

        iota16 = lax.iota(jnp.int32, L)

        def compute(c):
            b = c % 2
            off = base + c * C
            for g in range(n_groups):
                sl = pl.ds(g * L, L)
                rows = g * L + iota16
                nb_f = nb_v[b][sl].astype(jnp.float32)
                uoff = (uid_v[b][sl] & 1) * D

                accs0 = tuple(jnp.zeros((L,), jnp.float32)
                              for _ in range(JMAX))

                def d_body(dd, accs, b=b, rows=rows, uoff=uoff):
                    col = (dd & (D - L)) + ((iota16 + dd) & (L - 1))
                    u_val = plsc.load_gather(u_v[b], [rows, uoff + col])
                    new = []
                    for j in range(JMAX):
                        e_val = plsc.load_gather(
                            e_v[b], [rows, col + (j + 1) * D])
                        diff = u_val - e_val
                        new.append(accs[j] + diff * diff)
                    return tuple(new)

                accs = lax.fori_loop(0, D, d_body, accs0)

                inf = jnp.full((L,), jnp.inf, jnp.float32)
                best = inf
                bj = jnp.zeros((L,), jnp.float32)
                for j in range(JMAX):
                    jj = float(j + 1)
                    dj = jnp.where(nb_f >= jj, accs[j], inf)
                    upd = dj < best
                    best = jnp.where(upd, dj, best)
                    bj = jnp.where(upd, jj, bj)
                resp = (bj - 1.0) / (nb_f - 1.0) + 1.0
                resp_v[b][sl] = resp
            return pltpu.async_copy(
                resp_v[b], out_hbm.at[pl.ds(off, C)], osem)

        out_hs = []
        pend = fire(0)
        for c in range(n_chunks):
            nxt = fire(c + 1) if c + 1 < n_chunks else None
            drain(c, pend)
            if c >= 2:
                out_hs[c - 2].wait()
            out_hs.append(compute(c))
            pend = nxt
        for h in out_hs[-2:]:
            h.wait()

    return kern


def kernel(user_ids, item_ids, concept_ids, users_w, item_resp_w,
           nb_modalities, mask):
    B = user_ids.shape[0]
    ITEM_N, M = mask.shape
    USER_N, D = users_w.shape
    users2 = users_w.reshape(USER_N // 2, 2 * D)
    items896 = item_resp_w.reshape(ITEM_N, M * D)
    kern = _build(B, ITEM_N, USER_N, M, D, 16)
    return kern(user_ids, item_ids, users2, items896, nb_modalities)


# speedup vs baseline: 1.0012x; 1.0012x over previous
import functools

import jax
import jax.numpy as jnp
from jax import lax
from jax.experimental import pallas as pl
from jax.experimental.pallas import tpu as pltpu
from jax.experimental.pallas import tpu_sc as plsc

NC = 2
NS = 16
L = 16
NW = NC * NS


@functools.lru_cache(maxsize=None)
def _build(B, ITEM_N, USER_N, M, D, C):
    per_w = B // NW
    n_chunks = per_w // C
    n_groups = C // L
    JMAX = M - 2

    mesh = plsc.VectorSubcoreMesh(core_axis_name="c", subcore_axis_name="s")

    @functools.partial(
        pl.kernel,
        out_type=jax.ShapeDtypeStruct((B,), jnp.float32),
        mesh=mesh,
        compiler_params=pltpu.CompilerParams(
            needs_layout_passes=False, use_tc_tiling_on_sc=True,
            skip_device_barrier=True),
        scratch_types=[
            [pltpu.VMEM((C + L,), jnp.int32) for _ in range(2)],
            [pltpu.VMEM((C + L,), jnp.int32) for _ in range(2)],
            [pltpu.VMEM((C, M * D), jnp.float32) for _ in range(2)],
            [pltpu.VMEM((C,), jnp.int32) for _ in range(2)],
            [pltpu.VMEM((C, 2 * D), jnp.float32) for _ in range(2)],
            [pltpu.VMEM((C,), jnp.int32) for _ in range(2)],
            [pltpu.VMEM((C,), jnp.float32) for _ in range(2)],
            [pltpu.SemaphoreType.DMA for _ in range(2)],
            pltpu.SemaphoreType.DMA,
        ],
    )
    def kern(uid_hbm, iid_hbm, users_hbm, items_hbm, nb_hbm, out_hbm,
             iid_v, uid_v, e_v, upix_v, u_v, nb_v, resp_v, sems, osem):
        wid = lax.axis_index("s") * NC + lax.axis_index("c")
        base = wid * per_w

        def fire(c):
            b = c % 2
            off = base + c * C
            pltpu.sync_copy(iid_hbm.at[pl.ds(off, C)],
                            iid_v[b].at[pl.ds(0, C)])
            pltpu.sync_copy(uid_hbm.at[pl.ds(off, C)],
                            uid_v[b].at[pl.ds(0, C)])

            pltpu.async_copy(
                items_hbm.at[iid_v[b].at[pl.ds(0, C)]], e_v[b], sems[b])
            for g in range(n_groups):
                sl = pl.ds(g * L, L)
                upix_v[b][sl] = lax.shift_right_logical(uid_v[b][sl], 1)
            pltpu.async_copy(
                users_hbm.at[upix_v[b]], u_v[b], sems[b])
            return pltpu.async_copy(
                nb_hbm.at[iid_v[b].at[pl.ds(0, C)]], nb_v[b], sems[b])

        def drain(c, nb_h):
            b = c % 2
            pltpu.make_async_copy(
                items_hbm.at[pl.ds(0, C), :], e_v[b], sems[b]).wait()
            pltpu.make_async_copy(
                users_hbm.at[pl.ds(0, C), :], u_v[b], sems[b]).wait()
            nb_h.wait()

        iota16 = lax.iota(jnp.int32, L)

        def compute(c):
            b = c % 2
            off = base + c * C
            for g in range(n_groups):
                sl = pl.ds(g * L, L)
                rows = g * L + iota16
                nb_f = nb_v[b][sl].astype(jnp.float32)
                uoff = (uid_v[b][sl] & 1) * D

                accs0 = tuple(jnp.zeros((L,), jnp.float32)
                              for _ in range(JMAX))

                def d_body(dd, accs, b=b, rows=rows, uoff=uoff):
                    col = (dd & (D - L)) + ((iota16 + dd) & (L - 1))
                    u_val = plsc.load_gather(u_v[b], [rows, uoff + col])
                    new = []
                    for j in range(JMAX):
                        e_val = plsc.load_gather(
                            e_v[b], [rows, col + (j + 1) * D])
                        diff = u_val - e_val
                        new.append(accs[j] + diff * diff)
                    return tuple(new)

                accs = lax.fori_loop(0, D, d_body, accs0)

                inf = jnp.full((L,), jnp.inf, jnp.float32)
                best = inf
                bj = jnp.zeros((L,), jnp.float32)
                for j in range(JMAX):
                    jj = float(j + 1)
                    dj = jnp.where(nb_f >= jj, accs[j], inf)
                    upd = dj < best
                    best = jnp.where(upd, dj, best)
                    bj = jnp.where(upd, jj, bj)
                resp = (bj - 1.0) / (nb_f - 1.0) + 1.0
                resp_v[b][sl] = resp
            return pltpu.async_copy(
                resp_v[b], out_hbm.at[pl.ds(off, C)], osem)

        out_hs = []
        pend = fire(0)
        for c in range(n_chunks):
            nxt = fire(c + 1) if c + 1 < n_chunks else None
            drain(c, pend)
            if c >= 2:
                out_hs[c - 2].wait()
            out_hs.append(compute(c))
            pend = nxt
        for h in out_hs[-2:]:
            h.wait()

    return kern


def kernel(user_ids, item_ids, concept_ids, users_w, item_resp_w,
           nb_modalities, mask):
    B = user_ids.shape[0]
    ITEM_N, M = mask.shape
    USER_N, D = users_w.shape
    users2 = users_w.reshape(USER_N // 2, 2 * D)
    one = (nb_modalities[0] * 0 + 1).astype(item_resp_w.dtype)
    items896 = item_resp_w.reshape(ITEM_N, M * D) * one
    kern = _build(B, ITEM_N, USER_N, M, D, 16)
    return kern(user_ids, item_ids, users2, items896, nb_modalities)


# speedup vs baseline: 1.5045x; 1.5027x over previous
import functools

import jax
import jax.numpy as jnp
from jax import lax
from jax.experimental import pallas as pl
from jax.experimental.pallas import tpu as pltpu
from jax.experimental.pallas import tpu_sc as plsc

NC = 2
NS = 16
L = 16
NW = NC * NS


@functools.lru_cache(maxsize=None)
def _build(B, ITEM_N, USER_N, M, D, C):
    per_w = B // NW
    n_chunks = per_w // C
    n_groups = C // L
    JMAX = M - 2
    EW = 24

    mesh = plsc.VectorSubcoreMesh(core_axis_name="c", subcore_axis_name="s")

    @functools.partial(
        pl.kernel,
        out_type=jax.ShapeDtypeStruct((B,), jnp.float32),
        mesh=mesh,
        compiler_params=pltpu.CompilerParams(
            needs_layout_passes=False, use_tc_tiling_on_sc=True,
            skip_device_barrier=True),
        scratch_types=[
            [pltpu.VMEM((C + L,), jnp.int32) for _ in range(2)],
            [pltpu.VMEM((C + L,), jnp.int32) for _ in range(2)],
            [pltpu.VMEM((EW * C, D), jnp.float32) for _ in range(2)],
            [pltpu.VMEM((C,), jnp.int32) for _ in range(2)],
            [pltpu.VMEM((C, 2 * D), jnp.float32) for _ in range(2)],
            [pltpu.VMEM((C,), jnp.int32) for _ in range(2)],
            [pltpu.VMEM((C,), jnp.float32) for _ in range(2)],
            [pltpu.SemaphoreType.DMA for _ in range(2)],
            pltpu.SemaphoreType.DMA,
        ],
    )
    def kern(uid_hbm, iid_hbm, users_hbm, items_hbm, nb_hbm, out_hbm,
             iid_v, uid_v, e_v, upix_v, u_v, nb_v, resp_v, sems, osem):
        wid = lax.axis_index("s") * NC + lax.axis_index("c")
        base = wid * per_w

        def fire(c):
            b = c % 2
            off = base + c * C
            pltpu.sync_copy(iid_hbm.at[pl.ds(off, C)],
                            iid_v[b].at[pl.ds(0, C)])
            pltpu.sync_copy(uid_hbm.at[pl.ds(off, C)],
                            uid_v[b].at[pl.ds(0, C)])

            def issue(i, carry):
                q = iid_v[b][pl.ds(i, L)][0] * M
                a = pl.multiple_of(q & -8, 8)
                pltpu.async_copy(
                    items_hbm.at[pl.ds(a, EW), :],
                    e_v[b].at[pl.ds(i * EW, EW), :], sems[b])
                return carry

            lax.fori_loop(0, C, issue, 0)
            for g in range(n_groups):
                sl = pl.ds(g * L, L)
                upix_v[b][sl] = lax.shift_right_logical(uid_v[b][sl], 1)
            pltpu.async_copy(
                users_hbm.at[upix_v[b]], u_v[b], sems[b])
            return pltpu.async_copy(
                nb_hbm.at[iid_v[b].at[pl.ds(0, C)]], nb_v[b], sems[b])

        def drain(c, nb_h):
            b = c % 2
            pltpu.make_async_copy(
                items_hbm.at[pl.ds(0, EW * C), :], e_v[b], sems[b]).wait()
            pltpu.make_async_copy(
                users_hbm.at[pl.ds(0, C), :], u_v[b], sems[b]).wait()
            nb_h.wait()

        iota16 = lax.iota(jnp.int32, L)

        def compute(c):
            b = c % 2
            off = base + c * C
            for g in range(n_groups):
                sl = pl.ds(g * L, L)
                rows = g * L + iota16
                nb_f = nb_v[b][sl].astype(jnp.float32)
                qm = iid_v[b][sl] * M
                woff = (qm & 7) + 1
                rowbase = rows * EW + woff
                prows = [rowbase + j for j in range(JMAX)]
                uoff = (uid_v[b][sl] & 1) * D

                accs0 = tuple(jnp.zeros((L,), jnp.float32)
                              for _ in range(JMAX))

                def d_body(dd, accs, b=b, rows=rows, uoff=uoff,
                           prows=prows):
                    col = (dd & (D - L)) + ((iota16 + dd) & (L - 1))
                    u_val = plsc.load_gather(u_v[b], [rows, uoff + col])
                    new = []
                    for j in range(JMAX):
                        e_val = plsc.load_gather(e_v[b], [prows[j], col])
                        diff = u_val - e_val
                        new.append(accs[j] + diff * diff)
                    return tuple(new)

                accs = lax.fori_loop(0, D, d_body, accs0)

                inf = jnp.full((L,), jnp.inf, jnp.float32)
                best = inf
                bj = jnp.zeros((L,), jnp.float32)
                for j in range(JMAX):
                    jj = float(j + 1)
                    dj = jnp.where(nb_f >= jj, accs[j], inf)
                    upd = dj < best
                    best = jnp.where(upd, dj, best)
                    bj = jnp.where(upd, jj, bj)
                resp = (bj - 1.0) / (nb_f - 1.0) + 1.0
                resp_v[b][sl] = resp
            return pltpu.async_copy(
                resp_v[b], out_hbm.at[pl.ds(off, C)], osem)

        out_hs = []
        pend = fire(0)
        for c in range(n_chunks):
            nxt = fire(c + 1) if c + 1 < n_chunks else None
            drain(c, pend)
            if c >= 2:
                out_hs[c - 2].wait()
            out_hs.append(compute(c))
            pend = nxt
        for h in out_hs[-2:]:
            h.wait()

    return kern


def kernel(user_ids, item_ids, concept_ids, users_w, item_resp_w,
           nb_modalities, mask):
    B = user_ids.shape[0]
    ITEM_N, M = mask.shape
    USER_N, D = users_w.shape
    users2 = users_w.reshape(USER_N // 2, 2 * D)
    kern = _build(B, ITEM_N, USER_N, M, D, 16)
    return kern(user_ids, item_ids, users2, item_resp_w, nb_modalities)
